# baseline (speedup 1.0000x reference)
import functools

import jax
import jax.numpy as jnp
from jax import lax
from jax.experimental import pallas as pl
from jax.experimental.pallas import tpu as pltpu
from jax.experimental.pallas import tpu_sc as plsc

_NUM_CORES = 2
_NUM_SUBCORES = 16


@functools.lru_cache(maxsize=None)
def _make_gather(B, V, D):
    nw = _NUM_CORES * _NUM_SUBCORES
    bpw = B // nw
    ngrp = bpw // 16
    half = bpw // 2
    mesh = plsc.VectorSubcoreMesh(
        core_axis_name="c", subcore_axis_name="s",
        num_cores=_NUM_CORES, num_subcores=_NUM_SUBCORES)

    @functools.partial(
        pl.kernel,
        out_type=jax.ShapeDtypeStruct((B, D), jnp.float32),
        mesh=mesh,
        scratch_types=[
            pltpu.VMEM((bpw,), jnp.int32),
            pltpu.VMEM((bpw, D), jnp.float32),
            pltpu.VMEM_SHARED((_NUM_SUBCORES, half, D), jnp.float32),
            pltpu.SemaphoreType.DMA,
            pltpu.SemaphoreType.DMA,
        ],
    )
    def gather(labels_hbm, table_hbm, out_hbm, idx_v, rows_v, srows,
               sem_t, sem_s):
        cid = lax.axis_index("c")
        sid = lax.axis_index("s")
        wid = sid * _NUM_CORES + cid
        base = wid * bpw
        pltpu.sync_copy(labels_hbm.at[pl.ds(base, bpw)], idx_v)

        def fire2(g, carry):
            vec_e = idx_v[pl.ds(g * 32, 16)]
            vec_o = idx_v[pl.ds(g * 32 + 16, 16)]
            for k in range(16):
                pltpu.make_async_copy(
                    table_hbm.at[pl.ds(vec_e[k], 1)],
                    rows_v.at[pl.ds(g * 32 + k, 1)], sem_t).start()
                pltpu.make_async_copy(
                    table_hbm.at[pl.ds(vec_o[k], 1)],
                    srows.at[sid].at[pl.ds(g * 16 + k, 1)],
                    sem_s).start()
            return carry

        lax.fori_loop(0, ngrp // 2, fire2, 0)
        pltpu.make_async_copy(
            table_hbm.at[pl.ds(0, half)],
            rows_v.at[pl.ds(0, half)], sem_t).wait()
        pltpu.make_async_copy(
            table_hbm.at[pl.ds(0, half)],
            srows.at[sid], sem_s).wait()
        def pull(j, carry):
            pltpu.sync_copy(
                srows.at[sid].at[pl.ds(j * 16, 16)],
                rows_v.at[pl.ds(j * 32 + 16, 16)])
            return carry

        lax.fori_loop(0, ngrp // 2, pull, 0)
        pltpu.sync_copy(rows_v, out_hbm.at[pl.ds(base, bpw)])

    return gather


@jax.jit
def _embed(labels, table):
    (B,) = labels.shape
    V, D = table.shape
    return _make_gather(B, V, D)(labels, table)


def kernel(labels, train, table):
    return _embed(labels.astype(jnp.int32), table)
